# baseline (speedup 1.0000x reference)
import jax
import jax.numpy as jnp
from jax.experimental import pallas as pl
from jax.experimental.pallas import tpu as pltpu

_TOPK_RATIO = 0.1
_NH = 4


def _fold_kernel(q_w, q_b, k_w, k_b, v_w, v_b, in_w, in_b, out_w, out_b,
                 pw1, pb1, wq_o, bq_o, wk_o, bk_o, wv_o, bv_o, wop_o, bop_o):
    E = q_w.shape[1]
    dot = lambda a, b: jax.lax.dot(a, b, preferred_element_type=jnp.float32)
    in_q = in_w[:, :E]
    in_k = in_w[:, E:2 * E]
    in_v = in_w[:, 2 * E:]
    wq_o[...] = dot(q_w[...], in_q)
    bq_o[...] = dot(q_b[...], in_q) + in_b[:, :E]
    wk_o[...] = dot(k_w[...], in_k).astype(jnp.bfloat16)
    bk_o[...] = dot(k_b[...], in_k) + in_b[:, E:2 * E]
    wv_o[...] = dot(v_w[...], in_v).astype(jnp.bfloat16)
    bv_o[...] = dot(v_b[...], in_v) + in_b[:, 2 * E:]
    wop_o[...] = dot(out_w[...], pw1[...])
    bop_o[...] = dot(out_b[...], pw1[...]) + pb1[...]


def _main_kernel(v_ref, t_ref,
                 pre_w1, pre_b1, pre_w2, pre_b2, pre_w3t, pre_b3,
                 ft_w, ft_b, fp_w, fp_b, agg_wt, agg_b,
                 wq, bq, wk, bk, wv, bv,
                 wop, bop, post_w2, post_b2, post_w3t, post_b3,
                 out_ref, cam_s, kk_s, vv_s):
    f32 = jnp.float32
    bf16 = jnp.bfloat16
    dot = lambda a, b: jax.lax.dot(a, b, preferred_element_type=f32)
    dot_nt = lambda a, b: jax.lax.dot_general(
        a, b, (((1,), (1,)), ((), ())), preferred_element_type=f32)
    relu = lambda x: jnp.maximum(x, 0.0)

    b = pl.program_id(0)
    S = v_ref.shape[1]
    VD = v_ref.shape[2]
    E = wq.shape[1]
    HD = E // _NH
    k_sel = min(max(1, int(_TOPK_RATIO * S)), S)

    slot2 = jax.lax.rem(b + 1, 2)
    cam_p = cam_s[slot2]
    kk_p = kk_s[slot2]
    vv_p = vv_s[slot2]

    q = dot(cam_p, wq[...]) + bq[...]
    bmt = (jax.lax.broadcasted_iota(jnp.int32, (_NH, E), 1) // HD ==
           jax.lax.broadcasted_iota(jnp.int32, (_NH, E), 0)).astype(f32)
    bmq = bmt * q
    logits = dot_nt(bmq, kk_p) * (1.0 / jnp.sqrt(f32(HD)))
    logits = logits - jnp.max(logits, axis=1, keepdims=True)
    ee = jnp.exp(logits)
    probs = ee / jnp.sum(ee, axis=1, keepdims=True)
    ctx_full = dot(probs, vv_p)
    ctx = jnp.sum(ctx_full * bmt, axis=0, keepdims=True)

    g1 = relu(dot(ctx, wop[...]) + bop[...])
    g2 = relu(dot(g1, post_w2[...]) + post_b2[...])
    o = jnp.sum(g2 * post_w3t[...], axis=1, keepdims=True) + post_b3[...]
    out_ref[0] = jnp.broadcast_to(o, (1, S))

    vis_f = v_ref[0]
    txt_f = t_ref[0]

    h1 = relu(dot(vis_f, pre_w1[...]) + pre_b1[...])
    h2 = relu(dot(h1, pre_w2[...]) + pre_b2[...])
    s_row = dot_nt(pre_w3t[...], h2) + pre_b3[...]
    eye = (jax.lax.broadcasted_iota(jnp.int32, (S, S), 0) ==
           jax.lax.broadcasted_iota(jnp.int32, (S, S), 1)).astype(f32)
    s_col = dot_nt(eye, s_row)

    vis = vis_f.astype(bf16)
    txt = txt_f.astype(bf16)
    kk = dot(vis, wk[:VD]) + dot(txt, wk[VD:]) + bk[...]
    vv = dot(vis, wv[:VD]) + dot(txt, wv[VD:]) + bv[...]

    gt = (s_col > s_row).astype(f32) * (1.0 - eye)
    rank = jnp.sum(gt, axis=0, keepdims=True)
    sel = rank < f32(k_sel)

    tt = relu(dot(vis, ft_w[...]) + ft_b[...])
    c = relu(dot(tt.astype(bf16), fp_w[...]) + fp_b[...])
    lg = dot_nt(agg_wt[...], c) + agg_b[...]
    lg = jnp.where(sel, lg, -jnp.inf)
    lg = lg - jnp.max(lg, axis=1, keepdims=True)
    e = jnp.exp(lg)
    aw = e / jnp.sum(e, axis=1, keepdims=True)
    cam = dot(aw, c)

    slot = jax.lax.rem(b, 2)
    cam_s[slot] = cam
    kk_s[slot] = kk
    vv_s[slot] = vv


def kernel(visual_feat, text_feat,
           pre_w1, pre_b1, pre_w2, pre_b2, pre_w3, pre_b3,
           ft_w, ft_b, fp_w, fp_b, agg_w, agg_b,
           q_w, q_b, k_w, k_b, v_w, v_b, in_w, in_b, out_w, out_b,
           post_w1, post_b1, post_w2, post_b2, post_w3, post_b3):
    B, S, VD = visual_feat.shape
    TD = text_feat.shape[2]
    E = q_w.shape[1]
    CAM = fp_w.shape[1]
    FD = VD + TD
    f32 = jnp.float32
    bf16 = jnp.bfloat16
    row = lambda x: x.reshape(1, -1)

    wq, bq, wk, bk, wv, bv, wop, bop = pl.pallas_call(
        _fold_kernel,
        out_shape=[
            jax.ShapeDtypeStruct((CAM, E), f32),
            jax.ShapeDtypeStruct((1, E), f32),
            jax.ShapeDtypeStruct((FD, E), bf16),
            jax.ShapeDtypeStruct((1, E), f32),
            jax.ShapeDtypeStruct((FD, E), bf16),
            jax.ShapeDtypeStruct((1, E), f32),
            jax.ShapeDtypeStruct((E, post_w1.shape[1]), f32),
            jax.ShapeDtypeStruct((1, post_w1.shape[1]), f32),
        ],
    )(q_w, row(q_b), k_w, row(k_b), v_w, row(v_b), in_w, row(in_b),
      out_w, row(out_b), post_w1, row(post_b1))

    full = lambda a: pl.BlockSpec(a.shape, lambda *_: (0,) * a.ndim)
    operands = (
        visual_feat, text_feat,
        pre_w1, row(pre_b1), pre_w2, row(pre_b2), row(pre_w3), row(pre_b3),
        ft_w.astype(bf16), row(ft_b), fp_w.astype(bf16), row(fp_b),
        row(agg_w), row(agg_b),
        wq, bq, wk, bk, wv, bv,
        wop, bop, post_w2, row(post_b2), row(post_w3), row(post_b3),
    )
    last = B - 1
    in_specs = [
        pl.BlockSpec((1, S, VD), lambda b: (jnp.minimum(b, last), 0, 0)),
        pl.BlockSpec((1, S, TD), lambda b: (jnp.minimum(b, last), 0, 0)),
    ] + [full(a) for a in operands[2:]]

    out3 = pl.pallas_call(
        _main_kernel,
        grid=(B + 1,),
        in_specs=in_specs,
        out_specs=pl.BlockSpec((1, 1, S), lambda b: (jnp.maximum(b - 1, 0), 0, 0)),
        out_shape=jax.ShapeDtypeStruct((B, 1, S), f32),
        scratch_shapes=[
            pltpu.VMEM((2, 1, CAM), f32),
            pltpu.VMEM((2, S, E), f32),
            pltpu.VMEM((2, S, E), f32),
        ],
        compiler_params=pltpu.CompilerParams(
            dimension_semantics=("arbitrary",),
            vmem_limit_bytes=56 * 1024 * 1024,
        ),
    )(*operands)
    return out3.reshape(B, S)


# speedup vs baseline: 1.0126x; 1.0126x over previous
import jax
import jax.numpy as jnp
from jax.experimental import pallas as pl
from jax.experimental.pallas import tpu as pltpu

_TOPK_RATIO = 0.1
_NH = 4


def _fold_kernel(q_w, q_b, k_w, k_b, v_w, v_b, in_w, in_b, out_w, out_b,
                 pw1, pb1, wq_o, bq_o, wk_o, bk_o, wv_o, bv_o, wop_o, bop_o):
    E = q_w.shape[1]
    dot = lambda a, b: jax.lax.dot(a, b, preferred_element_type=jnp.float32)
    in_q = in_w[:, :E]
    in_k = in_w[:, E:2 * E]
    in_v = in_w[:, 2 * E:]
    wq_o[...] = dot(q_w[...], in_q)
    bq_o[...] = dot(q_b[...], in_q) + in_b[:, :E]
    wk_o[...] = dot(k_w[...], in_k)
    bk_o[...] = dot(k_b[...], in_k) + in_b[:, E:2 * E]
    wv_o[...] = dot(v_w[...], in_v)
    bv_o[...] = dot(v_b[...], in_v) + in_b[:, 2 * E:]
    wop_o[...] = dot(out_w[...], pw1[...])
    bop_o[...] = dot(out_b[...], pw1[...]) + pb1[...]


def _main_kernel(v_ref, t_ref,
                 pre_w1, pre_b1, pre_w2, pre_b2, pre_w3t, pre_b3,
                 ft_w, ft_b, fp_w, fp_b, agg_wt, agg_b,
                 wq, bq, wk, bk, wv, bv,
                 wop, bop, post_w2, post_b2, post_w3t, post_b3,
                 out_ref, cam_s, kk_s, vv_s):
    f32 = jnp.float32
    bf16 = jnp.bfloat16
    dot = lambda a, b: jax.lax.dot(a, b, preferred_element_type=f32)
    dot_nt = lambda a, b: jax.lax.dot_general(
        a, b, (((1,), (1,)), ((), ())), preferred_element_type=f32)
    relu = lambda x: jnp.maximum(x, 0.0)

    b = pl.program_id(0)
    S = v_ref.shape[1]
    VD = v_ref.shape[2]
    E = wq.shape[1]
    HD = E // _NH
    k_sel = min(max(1, int(_TOPK_RATIO * S)), S)

    slot2 = jax.lax.rem(b + 1, 2)
    cam_p = cam_s[slot2]
    kk_p = kk_s[slot2]
    vv_p = vv_s[slot2]

    q = dot(cam_p, wq[...]) + bq[...]
    bmt = (jax.lax.broadcasted_iota(jnp.int32, (_NH, E), 1) // HD ==
           jax.lax.broadcasted_iota(jnp.int32, (_NH, E), 0)).astype(f32)
    bmq = bmt * q
    logits = dot_nt(bmq, kk_p) * (1.0 / jnp.sqrt(f32(HD)))
    logits = logits - jnp.max(logits, axis=1, keepdims=True)
    ee = jnp.exp(logits)
    probs = ee / jnp.sum(ee, axis=1, keepdims=True)
    ctx_full = dot(probs, vv_p)
    ctx = jnp.sum(ctx_full * bmt, axis=0, keepdims=True)

    g1 = relu(dot(ctx, wop[...]) + bop[...])
    g2 = relu(dot(g1, post_w2[...]) + post_b2[...])
    o = jnp.sum(g2 * post_w3t[...], axis=1, keepdims=True) + post_b3[...]
    out_ref[0] = jnp.broadcast_to(o, (1, S))

    vis_f = v_ref[0]
    txt_f = t_ref[0]

    h1 = relu(dot(vis_f, pre_w1[...]) + pre_b1[...])
    h2 = relu(dot(h1, pre_w2[...]) + pre_b2[...])
    s_row = dot_nt(pre_w3t[...], h2) + pre_b3[...]
    eye = (jax.lax.broadcasted_iota(jnp.int32, (S, S), 0) ==
           jax.lax.broadcasted_iota(jnp.int32, (S, S), 1)).astype(f32)
    s_col = dot_nt(eye, s_row)

    kk = dot(vis_f, wk[:VD]) + dot(txt_f, wk[VD:]) + bk[...]
    vv = dot(vis_f, wv[:VD]) + dot(txt_f, wv[VD:]) + bv[...]

    gt = (s_col > s_row).astype(f32) * (1.0 - eye)
    rank = jnp.sum(gt, axis=0, keepdims=True)
    sel = rank < f32(k_sel)

    tt = relu(dot(vis_f, ft_w[...]) + ft_b[...])
    c = relu(dot(tt, fp_w[...]) + fp_b[...])
    lg = dot_nt(agg_wt[...], c) + agg_b[...]
    lg = jnp.where(sel, lg, -jnp.inf)
    lg = lg - jnp.max(lg, axis=1, keepdims=True)
    e = jnp.exp(lg)
    aw = e / jnp.sum(e, axis=1, keepdims=True)
    cam = dot(aw, c)

    slot = jax.lax.rem(b, 2)
    cam_s[slot] = cam
    kk_s[slot] = kk
    vv_s[slot] = vv


def kernel(visual_feat, text_feat,
           pre_w1, pre_b1, pre_w2, pre_b2, pre_w3, pre_b3,
           ft_w, ft_b, fp_w, fp_b, agg_w, agg_b,
           q_w, q_b, k_w, k_b, v_w, v_b, in_w, in_b, out_w, out_b,
           post_w1, post_b1, post_w2, post_b2, post_w3, post_b3):
    B, S, VD = visual_feat.shape
    TD = text_feat.shape[2]
    E = q_w.shape[1]
    CAM = fp_w.shape[1]
    FD = VD + TD
    f32 = jnp.float32
    bf16 = jnp.bfloat16
    row = lambda x: x.reshape(1, -1)

    wq, bq, wk, bk, wv, bv, wop, bop = pl.pallas_call(
        _fold_kernel,
        out_shape=[
            jax.ShapeDtypeStruct((CAM, E), f32),
            jax.ShapeDtypeStruct((1, E), f32),
            jax.ShapeDtypeStruct((FD, E), f32),
            jax.ShapeDtypeStruct((1, E), f32),
            jax.ShapeDtypeStruct((FD, E), f32),
            jax.ShapeDtypeStruct((1, E), f32),
            jax.ShapeDtypeStruct((E, post_w1.shape[1]), f32),
            jax.ShapeDtypeStruct((1, post_w1.shape[1]), f32),
        ],
    )(q_w, row(q_b), k_w, row(k_b), v_w, row(v_b), in_w, row(in_b),
      out_w, row(out_b), post_w1, row(post_b1))

    full = lambda a: pl.BlockSpec(a.shape, lambda *_: (0,) * a.ndim)
    operands = (
        visual_feat, text_feat,
        pre_w1, row(pre_b1), pre_w2, row(pre_b2), row(pre_w3), row(pre_b3),
        ft_w, row(ft_b), fp_w, row(fp_b),
        row(agg_w), row(agg_b),
        wq, bq, wk, bk, wv, bv,
        wop, bop, post_w2, row(post_b2), row(post_w3), row(post_b3),
    )
    last = B - 1
    in_specs = [
        pl.BlockSpec((1, S, VD), lambda b: (jnp.minimum(b, last), 0, 0)),
        pl.BlockSpec((1, S, TD), lambda b: (jnp.minimum(b, last), 0, 0)),
    ] + [full(a) for a in operands[2:]]

    out3 = pl.pallas_call(
        _main_kernel,
        grid=(B + 1,),
        in_specs=in_specs,
        out_specs=pl.BlockSpec((1, 1, S), lambda b: (jnp.maximum(b - 1, 0), 0, 0)),
        out_shape=jax.ShapeDtypeStruct((B, 1, S), f32),
        scratch_shapes=[
            pltpu.VMEM((2, 1, CAM), f32),
            pltpu.VMEM((2, S, E), f32),
            pltpu.VMEM((2, S, E), f32),
        ],
        compiler_params=pltpu.CompilerParams(
            dimension_semantics=("arbitrary",),
            vmem_limit_bytes=56 * 1024 * 1024,
        ),
    )(*operands)
    return out3.reshape(B, S)


# speedup vs baseline: 1.0276x; 1.0149x over previous
import jax
import jax.numpy as jnp
from jax.experimental import pallas as pl
from jax.experimental.pallas import tpu as pltpu

_TOPK_RATIO = 0.1
_NH = 4


def _fold_kernel(q_w, q_b, k_w, k_b, v_w, v_b, in_w, in_b, out_w, out_b,
                 pw1, pb1, wq_o, bq_o, wk_o, bk_o, wv_o, bv_o, wop_o, bop_o):
    E = q_w.shape[1]
    dot = lambda a, b: jax.lax.dot(a, b, preferred_element_type=jnp.float32)
    in_q = in_w[:, :E]
    in_k = in_w[:, E:2 * E]
    in_v = in_w[:, 2 * E:]
    wq_o[...] = dot(q_w[...], in_q)
    bq_o[...] = dot(q_b[...], in_q) + in_b[:, :E]
    wk_o[...] = dot(k_w[...], in_k)
    bk_o[...] = dot(k_b[...], in_k) + in_b[:, E:2 * E]
    wv_o[...] = dot(v_w[...], in_v)
    bv_o[...] = dot(v_b[...], in_v) + in_b[:, 2 * E:]
    wop_o[...] = dot(out_w[...], pw1[...])
    bop_o[...] = dot(out_b[...], pw1[...]) + pb1[...]


def _main_kernel(v_ref, t_ref,
                 pre_w1, pre_b1, pre_w2, pre_b2, pre_w3t, pre_b3,
                 ft_w, ft_b, fp_w, fp_b, agg_wt, agg_b,
                 wq, bq, wk, bk, wv, bv,
                 wop, bop, post_w2, post_b2, post_w3t, post_b3,
                 out_ref, cam_s, kk_s, vv_s):
    f32 = jnp.float32
    bf16 = jnp.bfloat16
    dot = lambda a, b: jax.lax.dot(a, b, preferred_element_type=f32)
    dot_nt = lambda a, b: jax.lax.dot_general(
        a, b, (((1,), (1,)), ((), ())), preferred_element_type=f32)
    relu = lambda x: jnp.maximum(x, 0.0)

    b = pl.program_id(0)
    S = v_ref.shape[1]
    VD = v_ref.shape[2]
    E = wq.shape[1]
    HD = E // _NH
    k_sel = min(max(1, int(_TOPK_RATIO * S)), S)

    slot2 = jax.lax.rem(b + 1, 2)
    cam_p = cam_s[slot2]
    kk_p = kk_s[slot2]
    vv_p = vv_s[slot2]

    q = dot(cam_p, wq[...]) + bq[...]
    bmt = (jax.lax.broadcasted_iota(jnp.int32, (_NH, E), 1) // HD ==
           jax.lax.broadcasted_iota(jnp.int32, (_NH, E), 0)).astype(f32)
    bmq = bmt * q
    logits = dot_nt(bmq, kk_p) * (1.0 / jnp.sqrt(f32(HD)))
    logits = logits - jnp.max(logits, axis=1, keepdims=True)
    ee = jnp.exp(logits)
    probs = ee / jnp.sum(ee, axis=1, keepdims=True)
    ctx_full = dot(probs, vv_p)
    ctx = jnp.sum(ctx_full * bmt, axis=0, keepdims=True)

    g1 = relu(dot(ctx, wop[...]) + bop[...])
    g2 = relu(dot(g1, post_w2[...]) + post_b2[...])
    o = jnp.sum(g2 * post_w3t[...], axis=1, keepdims=True) + post_b3[...]
    out_ref[0] = jnp.broadcast_to(o, (1, S))

    vis_f = v_ref[0]
    txt_f = t_ref[0]

    h1 = relu(dot(vis_f, pre_w1[...]) + pre_b1[...])
    h2 = relu(dot(h1, pre_w2[...]) + pre_b2[...])
    s_row = dot_nt(pre_w3t[...], h2) + pre_b3[...]

    kk = dot(vis_f, wk[:VD]) + dot(txt_f, wk[VD:]) + bk[...]
    vv = dot(vis_f, wv[:VD]) + dot(txt_f, wv[VD:]) + bv[...]

    lo = jnp.min(s_row, axis=1, keepdims=True)
    hi = jnp.max(s_row, axis=1, keepdims=True)
    wfrac = jax.lax.broadcasted_iota(jnp.int32, (8, 1), 0).astype(f32) * 0.125
    for _ in range(13):
        ts = lo + (hi - lo) * wfrac
        cnt = jnp.sum((s_row >= ts).astype(f32), axis=1, keepdims=True)
        ok = cnt >= f32(k_sel)
        lo = jnp.max(jnp.where(ok, ts, -jnp.inf), axis=0, keepdims=True)
        hi = jnp.minimum(
            jnp.min(jnp.where(ok, jnp.inf, ts), axis=0, keepdims=True), hi)
    sel = s_row >= lo

    tt = relu(dot(vis_f, ft_w[...]) + ft_b[...])
    c = relu(dot(tt, fp_w[...]) + fp_b[...])
    lg = dot_nt(agg_wt[...], c) + agg_b[...]
    lg = jnp.where(sel, lg, -jnp.inf)
    lg = lg - jnp.max(lg, axis=1, keepdims=True)
    e = jnp.exp(lg)
    aw = e / jnp.sum(e, axis=1, keepdims=True)
    cam = dot(aw, c)

    slot = jax.lax.rem(b, 2)
    cam_s[slot] = cam
    kk_s[slot] = kk
    vv_s[slot] = vv


def kernel(visual_feat, text_feat,
           pre_w1, pre_b1, pre_w2, pre_b2, pre_w3, pre_b3,
           ft_w, ft_b, fp_w, fp_b, agg_w, agg_b,
           q_w, q_b, k_w, k_b, v_w, v_b, in_w, in_b, out_w, out_b,
           post_w1, post_b1, post_w2, post_b2, post_w3, post_b3):
    B, S, VD = visual_feat.shape
    TD = text_feat.shape[2]
    E = q_w.shape[1]
    CAM = fp_w.shape[1]
    FD = VD + TD
    f32 = jnp.float32
    bf16 = jnp.bfloat16
    row = lambda x: x.reshape(1, -1)

    wq, bq, wk, bk, wv, bv, wop, bop = pl.pallas_call(
        _fold_kernel,
        out_shape=[
            jax.ShapeDtypeStruct((CAM, E), f32),
            jax.ShapeDtypeStruct((1, E), f32),
            jax.ShapeDtypeStruct((FD, E), f32),
            jax.ShapeDtypeStruct((1, E), f32),
            jax.ShapeDtypeStruct((FD, E), f32),
            jax.ShapeDtypeStruct((1, E), f32),
            jax.ShapeDtypeStruct((E, post_w1.shape[1]), f32),
            jax.ShapeDtypeStruct((1, post_w1.shape[1]), f32),
        ],
    )(q_w, row(q_b), k_w, row(k_b), v_w, row(v_b), in_w, row(in_b),
      out_w, row(out_b), post_w1, row(post_b1))

    full = lambda a: pl.BlockSpec(a.shape, lambda *_: (0,) * a.ndim)
    operands = (
        visual_feat, text_feat,
        pre_w1, row(pre_b1), pre_w2, row(pre_b2), row(pre_w3), row(pre_b3),
        ft_w, row(ft_b), fp_w, row(fp_b),
        row(agg_w), row(agg_b),
        wq, bq, wk, bk, wv, bv,
        wop, bop, post_w2, row(post_b2), row(post_w3), row(post_b3),
    )
    last = B - 1
    in_specs = [
        pl.BlockSpec((1, S, VD), lambda b: (jnp.minimum(b, last), 0, 0)),
        pl.BlockSpec((1, S, TD), lambda b: (jnp.minimum(b, last), 0, 0)),
    ] + [full(a) for a in operands[2:]]

    out3 = pl.pallas_call(
        _main_kernel,
        grid=(B + 1,),
        in_specs=in_specs,
        out_specs=pl.BlockSpec((1, 1, S), lambda b: (jnp.maximum(b - 1, 0), 0, 0)),
        out_shape=jax.ShapeDtypeStruct((B, 1, S), f32),
        scratch_shapes=[
            pltpu.VMEM((2, 1, CAM), f32),
            pltpu.VMEM((2, S, E), f32),
            pltpu.VMEM((2, S, E), f32),
        ],
        compiler_params=pltpu.CompilerParams(
            dimension_semantics=("arbitrary",),
            vmem_limit_bytes=56 * 1024 * 1024,
        ),
    )(*operands)
    return out3.reshape(B, S)
